# baseline (speedup 1.0000x reference)
import functools

import jax
import jax.numpy as jnp
from jax import lax
from jax.experimental import pallas as pl
from jax.experimental.pallas import tpu as pltpu
from jax.experimental.pallas import tpu_sc as plsc

N = 10000
E = 320000
D = 128
H = 128
B = 64
K = 30
C = 10
CONV_OUT = 32
KW = 5
T = K - KW + 1

NP_ = 10240
NPAD_ROWS = NP_ - N
NC, NS, L = 2, 16, 16
CH = 128
UNITS_PER_TILE = 80
EP = NC * NS * UNITS_PER_TILE * CH
ROWS_PER_TILE = NP_ // NS

NEG = -1.0e30


@functools.lru_cache(maxsize=None)
def _mesh():
  return plsc.VectorSubcoreMesh(
      core_axis_name="c", subcore_axis_name="s", num_cores=NC,
      num_subcores=NS)


def _make_agg(with_deg):
  out_type = jax.ShapeDtypeStruct((NC, NP_, D), jnp.float32)
  if with_deg:
    out_type = [out_type, jax.ShapeDtypeStruct((NC * NP_,), jnp.float32)]
  scratch = [
      pltpu.VMEM_SHARED((NP_, D), jnp.float32),
      pltpu.VMEM((UNITS_PER_TILE // 2, CH), jnp.int32),
      pltpu.VMEM((UNITS_PER_TILE // 2, CH), jnp.int32),
      pltpu.VMEM((CH, D), jnp.float32),
      pltpu.VMEM((CH, D), jnp.float32),
      pltpu.SemaphoreType.DMA,
      pltpu.SemaphoreType.DMA,
      pltpu.SemaphoreType.DMA,
      pltpu.SemaphoreType.DMA,
  ]
  if with_deg:
    scratch.insert(1, pltpu.VMEM_SHARED((NP_,), jnp.float32))
    scratch.append(pltpu.VMEM((CH,), jnp.float32))
    scratch.append(pltpu.SemaphoreType.DMA)

  def body(h_hbm, src_hbm, dst_hbm, zeros_hbm, zcol_hbm, *rest):
    if with_deg:
      (out_hbm, deg_hbm, acc, dega, src_v, dst_v, rows_a, rows_b,
       sem_a, sem_b, ssem_a, ssem_b, ones_v, dsem) = rest
    else:
      (out_hbm, acc, src_v, dst_v, rows_a, rows_b, sem_a, sem_b,
       ssem_a, ssem_b) = rest
    c = lax.axis_index("c")
    s = lax.axis_index("s")
    w = c * NS + s

    pltpu.sync_copy(zeros_hbm.at[pl.ds(s * ROWS_PER_TILE, ROWS_PER_TILE)],
                    acc.at[pl.ds(s * ROWS_PER_TILE, ROWS_PER_TILE)])
    if with_deg:
      pltpu.sync_copy(zcol_hbm.at[pl.ds(s * ROWS_PER_TILE, ROWS_PER_TILE)],
                      dega.at[pl.ds(s * ROWS_PER_TILE, ROWS_PER_TILE)])
      for j in range(CH // L):
        ones_v[pl.ds(j * L, L)] = jnp.full((L,), 1.0, jnp.float32)
    plsc.subcore_barrier()

    UPH = UNITS_PER_TILE // 2

    for ph in range(2):
      pltpu.sync_copy(src_hbm.at[pl.ds(w * UNITS_PER_TILE + ph * UPH, UPH)],
                      src_v)
      pltpu.sync_copy(dst_hbm.at[pl.ds(w * UNITS_PER_TILE + ph * UPH, UPH)],
                      dst_v)
      pltpu.async_copy(h_hbm.at[src_v.at[0]], rows_a, sem_a)

      def pair(p, carry):
        u0 = 2 * p
        pltpu.async_copy(h_hbm.at[src_v.at[u0 + 1]], rows_b, sem_b)
        pltpu.make_async_copy(h_hbm.at[src_v.at[u0]], rows_a, sem_a).wait()
        pltpu.async_copy(rows_a, acc.at[dst_v.at[u0]], ssem_a, add=True)
        if with_deg:
          pltpu.async_copy(ones_v, dega.at[dst_v.at[u0]], dsem, add=True)
        pltpu.make_async_copy(h_hbm.at[src_v.at[u0 + 1]], rows_b,
                              sem_b).wait()
        pltpu.async_copy(rows_b, acc.at[dst_v.at[u0 + 1]], ssem_b, add=True)
        if with_deg:
          pltpu.async_copy(ones_v, dega.at[dst_v.at[u0 + 1]], dsem,
                           add=True)
        pltpu.make_async_copy(rows_a, acc.at[dst_v.at[u0]], ssem_a).wait()

        @pl.when(p < UPH // 2 - 1)
        def _():
          pltpu.async_copy(h_hbm.at[src_v.at[u0 + 2]], rows_a, sem_a)

        pltpu.make_async_copy(rows_b, acc.at[dst_v.at[u0 + 1]],
                              ssem_b).wait()
        return carry

      lax.fori_loop(0, UPH // 2, pair, 0)

      if with_deg:
        def deg_drain(i, carry):
          pltpu.make_async_copy(ones_v, dega.at[dst_v.at[0]], dsem).wait()
          return carry
        lax.fori_loop(0, UPH, deg_drain, 0)
    plsc.subcore_barrier()

    pltpu.sync_copy(acc.at[pl.ds(s * ROWS_PER_TILE, ROWS_PER_TILE)],
                    out_hbm.at[c, pl.ds(s * ROWS_PER_TILE, ROWS_PER_TILE)])
    if with_deg:
      pltpu.sync_copy(
          dega.at[pl.ds(s * ROWS_PER_TILE, ROWS_PER_TILE)],
          deg_hbm.at[pl.ds(c * NP_ + s * ROWS_PER_TILE, ROWS_PER_TILE)])

  return pl.kernel(body, out_type=out_type, mesh=_mesh(),
                   scratch_types=scratch,
                   compiler_params=pltpu.CompilerParams(
                       needs_layout_passes=False))


_agg_deg = functools.lru_cache(maxsize=None)(lambda: _make_agg(True))
_agg = functools.lru_cache(maxsize=None)(lambda: _make_agg(False))


_RB = 2048


def _layer1_body(parts, degp, h, Wl, Wr, bl, out, inv_out):
  i = pl.program_id(0)
  d = degp[0] + degp[1]
  inv = 1.0 / jnp.maximum(d, 1.0)
  inv_out[...] = inv
  _layer_common(parts, inv, h, Wl, Wr, bl, out, i)


def _layerN_body(parts, inv_ref, h, Wl, Wr, bl, out):
  i = pl.program_id(0)
  _layer_common(parts, inv_ref[...], h, Wl, Wr, bl, out, i)


def _layer_common(parts, inv, h, Wl, Wr, bl, out, i):
  pp = parts[0] + parts[1]
  aggn = pp * inv
  hn = (jnp.dot(aggn, Wl[...], preferred_element_type=jnp.float32)
        + jnp.dot(h[...], Wr[...], preferred_element_type=jnp.float32)
        + bl[...])
  hn = jnp.maximum(hn, 0.0)
  rows = i * _RB + lax.broadcasted_iota(jnp.int32, (_RB, D), 0)
  out[...] = jnp.where(rows < N, hn, 0.0)


def _layer1_tc(parts, degp, h, Wl, Wr, bl):
  grid = (NP_ // _RB,)
  return pl.pallas_call(
      _layer1_body,
      grid=grid,
      in_specs=[
          pl.BlockSpec((NC, _RB, D), lambda i: (0, i, 0)),
          pl.BlockSpec((NC, _RB, 1), lambda i: (0, i, 0)),
          pl.BlockSpec((_RB, D), lambda i: (i, 0)),
          pl.BlockSpec((D, H), lambda i: (0, 0)),
          pl.BlockSpec((D, H), lambda i: (0, 0)),
          pl.BlockSpec((1, H), lambda i: (0, 0)),
      ],
      out_specs=[
          pl.BlockSpec((_RB, H), lambda i: (i, 0)),
          pl.BlockSpec((_RB, 1), lambda i: (i, 0)),
      ],
      out_shape=[
          jax.ShapeDtypeStruct((NP_, H), jnp.float32),
          jax.ShapeDtypeStruct((NP_, 1), jnp.float32),
      ],
  )(parts, degp, h, Wl, Wr, bl)


def _layerN_tc(parts, inv, h, Wl, Wr, bl):
  grid = (NP_ // _RB,)
  return pl.pallas_call(
      _layerN_body,
      grid=grid,
      in_specs=[
          pl.BlockSpec((NC, _RB, D), lambda i: (0, i, 0)),
          pl.BlockSpec((_RB, 1), lambda i: (i, 0)),
          pl.BlockSpec((_RB, D), lambda i: (i, 0)),
          pl.BlockSpec((D, H), lambda i: (0, 0)),
          pl.BlockSpec((D, H), lambda i: (0, 0)),
          pl.BlockSpec((1, H), lambda i: (0, 0)),
      ],
      out_specs=pl.BlockSpec((_RB, H), lambda i: (i, 0)),
      out_shape=jax.ShapeDtypeStruct((NP_, H), jnp.float32),
  )(parts, inv, h, Wl, Wr, bl)


CHKR = 128
NB_BATCH = N // L


def _sortpool_body(h_hbm, batch_hbm, pooled_hbm,
                   bat_v, keys_v, stage_v, selidx_v, outrows_v, cnt_v, sem):
  c = lax.axis_index("c")
  s = lax.axis_index("s")
  w = c * NS + s

  pltpu.sync_copy(batch_hbm, bat_v)
  iota = lax.iota(jnp.int32, L)

  for gl in range(2):
    g = w * 2 + gl

    zz = jnp.zeros((L,), jnp.int32)
    cnt_v[0, :] = zz
    cnt_v[1, :] = zz

    def cnt_body(i, carry):
      v = bat_v[pl.ds(i * L, L)]
      cnt_v[0, :] = cnt_v[0, :] + jnp.where(v < g, 1, 0)
      cnt_v[1, :] = cnt_v[1, :] + jnp.where(v == g, 1, 0)
      return carry

    lax.fori_loop(0, NB_BATCH, cnt_body, 0)
    start = jnp.sum(cnt_v[0, :])
    cnt = jnp.sum(cnt_v[1, :])

    abase = pl.multiple_of((start // 8) * 8, 8)
    off = start - abase
    total = off + cnt
    nchunks = (total + CHKR - 1) // CHKR
    col16 = jnp.full((L,), D - 1, jnp.int32)

    def chunk_body(j, carry):
      pltpu.sync_copy(
          h_hbm.at[pl.ds(pl.multiple_of(abase + j * CHKR, 8), CHKR)], stage_v)
      for jj in range(CHKR // L):
        rows16 = iota + jj * L
        kv = plsc.load_gather(stage_v, [rows16, col16])
        pos = j * CHKR + jj * L + iota
        kv = jnp.where((pos >= off) & (pos < total), kv, NEG)
        keys_v[pl.ds(j * CHKR + jj * L, L)] = kv
      return carry

    lax.fori_loop(0, nchunks, chunk_body, 0)

    nv = (total + L - 1) // L

    selidx_v[pl.ds(gl * 2 * L, L)] = N + ((w * 29 + iota) % NPAD_ROWS)
    selidx_v[pl.ds(gl * 2 * L + L, L)] = N + ((w * 29 + 101 + iota)
                                              % NPAD_ROWS)

    def sel_body(k, carry):
      def mx_body(i, m):
        return jnp.maximum(m, jnp.max(keys_v[pl.ds(i * L, L)]))
      m = lax.fori_loop(0, nv, mx_body, jnp.float32(NEG))

      def fi_body(i, f):
        v = keys_v[pl.ds(i * L, L)]
        cand = jnp.where(v == m, iota + i * L, jnp.int32(2**30))
        return jnp.minimum(f, jnp.min(cand))
      found = lax.fori_loop(0, nv, fi_body, jnp.int32(2**30))

      node = jnp.where(k < cnt, abase + found,
                       N + ((w * 53 + k * 3) % NPAD_ROWS))
      plsc.store_scatter(keys_v, [jnp.full((L,), found, jnp.int32)],
                         jnp.full((L,), NEG, jnp.float32),
                         mask=(iota == 0))
      plsc.store_scatter(selidx_v, [jnp.full((L,), gl * 2 * L + k,
                                             jnp.int32)],
                         jnp.full((L,), node, jnp.int32),
                         mask=(iota == 0))
      return carry

    lax.fori_loop(0, K, sel_body, 0)

    pltpu.async_copy(h_hbm.at[selidx_v.at[pl.ds(gl * 2 * L, 2 * L)]],
                     outrows_v, sem).wait()
    pltpu.sync_copy(outrows_v.at[pl.ds(0, K)], pooled_hbm.at[g])


@functools.lru_cache(maxsize=None)
def _sortpool():
  return pl.kernel(
      _sortpool_body,
      out_type=jax.ShapeDtypeStruct((B, K, D), jnp.float32),
      mesh=_mesh(),
      scratch_types=[
          pltpu.VMEM((N,), jnp.int32),
          pltpu.VMEM((NP_,), jnp.float32),
          pltpu.VMEM((CHKR, D), jnp.float32),
          pltpu.VMEM((2 * 2 * L,), jnp.int32),
          pltpu.VMEM((2 * L, D), jnp.float32),
          pltpu.VMEM((2, L), jnp.int32),
          pltpu.SemaphoreType.DMA,
      ],
      compiler_params=pltpu.CompilerParams(needs_layout_passes=False),
  )


def _head_body(p2, WcT, bc2, W1r, b1r, W2, b2r, out):
  h1acc = jnp.zeros((B, H), jnp.float32)
  for t in range(T):
    zt = jnp.zeros((B, CONV_OUT), jnp.float32)
    for kw in range(KW):
      zt = zt + jnp.dot(p2[:, (t + kw) * H:(t + kw + 1) * H],
                        WcT[kw * H:(kw + 1) * H, :],
                        preferred_element_type=jnp.float32)
    a = jnp.maximum(zt + bc2[...], 0.0)
    h1acc = h1acc + jnp.dot(a, W1r[t * CONV_OUT:(t + 1) * CONV_OUT, :],
                            preferred_element_type=jnp.float32)
  h1 = jnp.maximum(h1acc + b1r[...], 0.0)
  logits = jnp.dot(h1, W2[...], preferred_element_type=jnp.float32) + b2r[...]
  m = jnp.max(logits, axis=-1, keepdims=True)
  out[...] = logits - m - jnp.log(
      jnp.sum(jnp.exp(logits - m), axis=-1, keepdims=True))


def _head_tc(p2, WcT, bc2, W1r, b1r, W2, b2r):
  return pl.pallas_call(
      _head_body,
      out_shape=jax.ShapeDtypeStruct((B, C), jnp.float32),
  )(p2, WcT, bc2, W1r, b1r, W2, b2r)


def kernel(x, edge_index, batch, Wl1, bl1, Wr1, Wl2, bl2, Wr2, Wl3, bl3, Wr3,
           Wc, bc, W1, b1, W2, b2):
  f32 = jnp.float32
  xp = jnp.zeros((NP_, D), f32).at[:N].set(x)
  src, dst = edge_index[0], edge_index[1]
  npad = EP - E
  pad_src = N + (jnp.arange(npad, dtype=jnp.int32) % NPAD_ROWS)
  pad_dst = N + ((jnp.arange(npad, dtype=jnp.int32) * 7) % NPAD_ROWS)
  src2d = jnp.concatenate([src, pad_src]).reshape(EP // CH, CH)
  dst2d = jnp.concatenate([dst, pad_dst]).reshape(EP // CH, CH)
  zeros = jnp.zeros((NP_, D), f32)
  zcol = jnp.zeros((NP_,), f32)

  parts, degp = _agg_deg()(xp, src2d, dst2d, zeros, zcol)
  h1, inv = _layer1_tc(parts, degp.reshape(NC, NP_, 1), xp, Wl1, Wr1,
                       bl1.reshape(1, H))
  parts2 = _agg()(h1, src2d, dst2d, zeros, zcol)
  h2 = _layerN_tc(parts2, inv, h1, Wl2, Wr2, bl2.reshape(1, H))
  parts3 = _agg()(h2, src2d, dst2d, zeros, zcol)
  h3 = _layerN_tc(parts3, inv, h2, Wl3, Wr3, bl3.reshape(1, H))

  pooled = _sortpool()(h3, batch)

  WcT = jnp.transpose(Wc, (2, 1, 0)).reshape(KW * H, CONV_OUT)
  W1r = W1.reshape(CONV_OUT, T, H).transpose(1, 0, 2).reshape(T * CONV_OUT, H)
  return _head_tc(pooled.reshape(B, K * D), WcT, bc.reshape(1, CONV_OUT),
                  W1r, b1.reshape(1, H), W2, b2.reshape(1, C))


# speedup vs baseline: 1.2047x; 1.2047x over previous
import functools

import jax
import jax.numpy as jnp
from jax import lax
from jax.experimental import pallas as pl
from jax.experimental.pallas import tpu as pltpu
from jax.experimental.pallas import tpu_sc as plsc

N = 10000
E = 320000
D = 128
H = 128
B = 64
K = 30
C = 10
CONV_OUT = 32
KW = 5
T = K - KW + 1

NP_ = 10240
NPAD_ROWS = NP_ - N
NC, NS, L = 2, 16, 16
CH = 128
UNITS_PER_TILE = 80
EP = NC * NS * UNITS_PER_TILE * CH
ROWS_PER_TILE = NP_ // NS

NEG = -1.0e30


@functools.lru_cache(maxsize=None)
def _mesh():
  return plsc.VectorSubcoreMesh(
      core_axis_name="c", subcore_axis_name="s", num_cores=NC,
      num_subcores=NS)


def _make_agg(with_deg):
  out_type = jax.ShapeDtypeStruct((NC, NP_, D), jnp.float32)
  if with_deg:
    out_type = [out_type, jax.ShapeDtypeStruct((NC * NP_,), jnp.float32)]
  scratch = [
      pltpu.VMEM_SHARED((NP_, D), jnp.float32),
      pltpu.VMEM((UNITS_PER_TILE // 2, CH), jnp.int32),
      pltpu.VMEM((UNITS_PER_TILE // 2, CH), jnp.int32),
      pltpu.VMEM((CH, D), jnp.float32),
      pltpu.VMEM((CH, D), jnp.float32),
      pltpu.SemaphoreType.DMA,
      pltpu.SemaphoreType.DMA,
      pltpu.SemaphoreType.DMA,
      pltpu.SemaphoreType.DMA,
  ]
  if with_deg:
    scratch.insert(1, pltpu.VMEM_SHARED((NP_,), jnp.float32))
    scratch.append(pltpu.VMEM((CH,), jnp.float32))
    scratch.append(pltpu.SemaphoreType.DMA)

  def body(h_hbm, src_hbm, dst_hbm, zeros_hbm, zcol_hbm, *rest):
    if with_deg:
      (out_hbm, deg_hbm, acc, dega, src_v, dst_v, rows_a, rows_b,
       sem_a, sem_b, ssem_a, ssem_b, ones_v, dsem) = rest
    else:
      (out_hbm, acc, src_v, dst_v, rows_a, rows_b, sem_a, sem_b,
       ssem_a, ssem_b) = rest
    c = lax.axis_index("c")
    s = lax.axis_index("s")
    w = c * NS + s

    pltpu.sync_copy(zeros_hbm.at[pl.ds(s * ROWS_PER_TILE, ROWS_PER_TILE)],
                    acc.at[pl.ds(s * ROWS_PER_TILE, ROWS_PER_TILE)])
    if with_deg:
      pltpu.sync_copy(zcol_hbm.at[pl.ds(s * ROWS_PER_TILE, ROWS_PER_TILE)],
                      dega.at[pl.ds(s * ROWS_PER_TILE, ROWS_PER_TILE)])
      for j in range(CH // L):
        ones_v[pl.ds(j * L, L)] = jnp.full((L,), 1.0, jnp.float32)
    plsc.subcore_barrier()

    UPH = UNITS_PER_TILE // 2

    for ph in range(2):
      pltpu.sync_copy(src_hbm.at[pl.ds(w * UNITS_PER_TILE + ph * UPH, UPH)],
                      src_v)
      pltpu.sync_copy(dst_hbm.at[pl.ds(w * UNITS_PER_TILE + ph * UPH, UPH)],
                      dst_v)
      pltpu.async_copy(h_hbm.at[src_v.at[0]], rows_a, sem_a)

      def pair(p, carry):
        u0 = 2 * p
        pltpu.async_copy(h_hbm.at[src_v.at[u0 + 1]], rows_b, sem_b)
        pltpu.make_async_copy(h_hbm.at[src_v.at[u0]], rows_a, sem_a).wait()
        pltpu.sync_copy(rows_a, acc.at[dst_v.at[u0]], add=True)
        if with_deg:
          pltpu.async_copy(ones_v, dega.at[dst_v.at[u0]], dsem, add=True)

        @pl.when(p < UPH // 2 - 1)
        def _():
          pltpu.async_copy(h_hbm.at[src_v.at[u0 + 2]], rows_a, sem_a)

        pltpu.make_async_copy(h_hbm.at[src_v.at[u0 + 1]], rows_b,
                              sem_b).wait()
        pltpu.sync_copy(rows_b, acc.at[dst_v.at[u0 + 1]], add=True)
        if with_deg:
          pltpu.async_copy(ones_v, dega.at[dst_v.at[u0 + 1]], dsem,
                           add=True)
        return carry

      lax.fori_loop(0, UPH // 2, pair, 0)

      if with_deg:
        def deg_drain(i, carry):
          pltpu.make_async_copy(ones_v, dega.at[dst_v.at[0]], dsem).wait()
          return carry
        lax.fori_loop(0, UPH, deg_drain, 0)
    plsc.subcore_barrier()

    pltpu.sync_copy(acc.at[pl.ds(s * ROWS_PER_TILE, ROWS_PER_TILE)],
                    out_hbm.at[c, pl.ds(s * ROWS_PER_TILE, ROWS_PER_TILE)])
    if with_deg:
      pltpu.sync_copy(
          dega.at[pl.ds(s * ROWS_PER_TILE, ROWS_PER_TILE)],
          deg_hbm.at[pl.ds(c * NP_ + s * ROWS_PER_TILE, ROWS_PER_TILE)])

  return pl.kernel(body, out_type=out_type, mesh=_mesh(),
                   scratch_types=scratch,
                   compiler_params=pltpu.CompilerParams(
                       needs_layout_passes=False))


_agg_deg = functools.lru_cache(maxsize=None)(lambda: _make_agg(True))
_agg = functools.lru_cache(maxsize=None)(lambda: _make_agg(False))


_RB = 2048


def _layer1_body(parts, degp, h, Wl, Wr, bl, out, inv_out):
  i = pl.program_id(0)
  d = degp[0] + degp[1]
  inv = 1.0 / jnp.maximum(d, 1.0)
  inv_out[...] = inv
  _layer_common(parts, inv, h, Wl, Wr, bl, out, i)


def _layerN_body(parts, inv_ref, h, Wl, Wr, bl, out):
  i = pl.program_id(0)
  _layer_common(parts, inv_ref[...], h, Wl, Wr, bl, out, i)


def _layer_common(parts, inv, h, Wl, Wr, bl, out, i):
  pp = parts[0] + parts[1]
  aggn = pp * inv
  hn = (jnp.dot(aggn, Wl[...], preferred_element_type=jnp.float32)
        + jnp.dot(h[...], Wr[...], preferred_element_type=jnp.float32)
        + bl[...])
  hn = jnp.maximum(hn, 0.0)
  rows = i * _RB + lax.broadcasted_iota(jnp.int32, (_RB, D), 0)
  out[...] = jnp.where(rows < N, hn, 0.0)


def _layer1_tc(parts, degp, h, Wl, Wr, bl):
  grid = (NP_ // _RB,)
  return pl.pallas_call(
      _layer1_body,
      grid=grid,
      in_specs=[
          pl.BlockSpec((NC, _RB, D), lambda i: (0, i, 0)),
          pl.BlockSpec((NC, _RB, 1), lambda i: (0, i, 0)),
          pl.BlockSpec((_RB, D), lambda i: (i, 0)),
          pl.BlockSpec((D, H), lambda i: (0, 0)),
          pl.BlockSpec((D, H), lambda i: (0, 0)),
          pl.BlockSpec((1, H), lambda i: (0, 0)),
      ],
      out_specs=[
          pl.BlockSpec((_RB, H), lambda i: (i, 0)),
          pl.BlockSpec((_RB, 1), lambda i: (i, 0)),
      ],
      out_shape=[
          jax.ShapeDtypeStruct((NP_, H), jnp.float32),
          jax.ShapeDtypeStruct((NP_, 1), jnp.float32),
      ],
  )(parts, degp, h, Wl, Wr, bl)


def _layerN_tc(parts, inv, h, Wl, Wr, bl):
  grid = (NP_ // _RB,)
  return pl.pallas_call(
      _layerN_body,
      grid=grid,
      in_specs=[
          pl.BlockSpec((NC, _RB, D), lambda i: (0, i, 0)),
          pl.BlockSpec((_RB, 1), lambda i: (i, 0)),
          pl.BlockSpec((_RB, D), lambda i: (i, 0)),
          pl.BlockSpec((D, H), lambda i: (0, 0)),
          pl.BlockSpec((D, H), lambda i: (0, 0)),
          pl.BlockSpec((1, H), lambda i: (0, 0)),
      ],
      out_specs=pl.BlockSpec((_RB, H), lambda i: (i, 0)),
      out_shape=jax.ShapeDtypeStruct((NP_, H), jnp.float32),
  )(parts, inv, h, Wl, Wr, bl)


CHKR = 128
NB_BATCH = N // L


def _sortpool_body(h_hbm, batch_hbm, pooled_hbm,
                   bat_v, keys_v, stage_v, selidx_v, outrows_v, cnt_v, sem):
  c = lax.axis_index("c")
  s = lax.axis_index("s")
  w = c * NS + s

  pltpu.sync_copy(batch_hbm, bat_v)
  iota = lax.iota(jnp.int32, L)

  for gl in range(2):
    g = w * 2 + gl

    zz = jnp.zeros((L,), jnp.int32)
    cnt_v[0, :] = zz
    cnt_v[1, :] = zz

    def cnt_body(i, carry):
      v = bat_v[pl.ds(i * L, L)]
      cnt_v[0, :] = cnt_v[0, :] + jnp.where(v < g, 1, 0)
      cnt_v[1, :] = cnt_v[1, :] + jnp.where(v == g, 1, 0)
      return carry

    lax.fori_loop(0, NB_BATCH, cnt_body, 0)
    start = jnp.sum(cnt_v[0, :])
    cnt = jnp.sum(cnt_v[1, :])

    abase = pl.multiple_of((start // 8) * 8, 8)
    off = start - abase
    total = off + cnt
    nchunks = (total + CHKR - 1) // CHKR
    col16 = jnp.full((L,), D - 1, jnp.int32)

    def chunk_body(j, carry):
      pltpu.sync_copy(
          h_hbm.at[pl.ds(pl.multiple_of(abase + j * CHKR, 8), CHKR)], stage_v)
      for jj in range(CHKR // L):
        rows16 = iota + jj * L
        kv = plsc.load_gather(stage_v, [rows16, col16])
        pos = j * CHKR + jj * L + iota
        kv = jnp.where((pos >= off) & (pos < total), kv, NEG)
        keys_v[pl.ds(j * CHKR + jj * L, L)] = kv
      return carry

    lax.fori_loop(0, nchunks, chunk_body, 0)

    nv = (total + L - 1) // L

    selidx_v[pl.ds(gl * 2 * L, L)] = N + ((w * 29 + iota) % NPAD_ROWS)
    selidx_v[pl.ds(gl * 2 * L + L, L)] = N + ((w * 29 + 101 + iota)
                                              % NPAD_ROWS)

    def sel_body(k, carry):
      def mx_body(i, m):
        return jnp.maximum(m, jnp.max(keys_v[pl.ds(i * L, L)]))
      m = lax.fori_loop(0, nv, mx_body, jnp.float32(NEG))

      def fi_body(i, f):
        v = keys_v[pl.ds(i * L, L)]
        cand = jnp.where(v == m, iota + i * L, jnp.int32(2**30))
        return jnp.minimum(f, jnp.min(cand))
      found = lax.fori_loop(0, nv, fi_body, jnp.int32(2**30))

      node = jnp.where(k < cnt, abase + found,
                       N + ((w * 53 + k * 3) % NPAD_ROWS))
      plsc.store_scatter(keys_v, [jnp.full((L,), found, jnp.int32)],
                         jnp.full((L,), NEG, jnp.float32),
                         mask=(iota == 0))
      plsc.store_scatter(selidx_v, [jnp.full((L,), gl * 2 * L + k,
                                             jnp.int32)],
                         jnp.full((L,), node, jnp.int32),
                         mask=(iota == 0))
      return carry

    lax.fori_loop(0, K, sel_body, 0)

    pltpu.async_copy(h_hbm.at[selidx_v.at[pl.ds(gl * 2 * L, 2 * L)]],
                     outrows_v, sem).wait()
    pltpu.sync_copy(outrows_v.at[pl.ds(0, K)], pooled_hbm.at[g])


@functools.lru_cache(maxsize=None)
def _sortpool():
  return pl.kernel(
      _sortpool_body,
      out_type=jax.ShapeDtypeStruct((B, K, D), jnp.float32),
      mesh=_mesh(),
      scratch_types=[
          pltpu.VMEM((N,), jnp.int32),
          pltpu.VMEM((NP_,), jnp.float32),
          pltpu.VMEM((CHKR, D), jnp.float32),
          pltpu.VMEM((2 * 2 * L,), jnp.int32),
          pltpu.VMEM((2 * L, D), jnp.float32),
          pltpu.VMEM((2, L), jnp.int32),
          pltpu.SemaphoreType.DMA,
      ],
      compiler_params=pltpu.CompilerParams(needs_layout_passes=False),
  )


def _head_body(p2, WcT, bc2, W1r, b1r, W2, b2r, out):
  h1acc = jnp.zeros((B, H), jnp.float32)
  for t in range(T):
    zt = jnp.zeros((B, CONV_OUT), jnp.float32)
    for kw in range(KW):
      zt = zt + jnp.dot(p2[:, (t + kw) * H:(t + kw + 1) * H],
                        WcT[kw * H:(kw + 1) * H, :],
                        preferred_element_type=jnp.float32)
    a = jnp.maximum(zt + bc2[...], 0.0)
    h1acc = h1acc + jnp.dot(a, W1r[t * CONV_OUT:(t + 1) * CONV_OUT, :],
                            preferred_element_type=jnp.float32)
  h1 = jnp.maximum(h1acc + b1r[...], 0.0)
  logits = jnp.dot(h1, W2[...], preferred_element_type=jnp.float32) + b2r[...]
  m = jnp.max(logits, axis=-1, keepdims=True)
  out[...] = logits - m - jnp.log(
      jnp.sum(jnp.exp(logits - m), axis=-1, keepdims=True))


def _head_tc(p2, WcT, bc2, W1r, b1r, W2, b2r):
  return pl.pallas_call(
      _head_body,
      out_shape=jax.ShapeDtypeStruct((B, C), jnp.float32),
  )(p2, WcT, bc2, W1r, b1r, W2, b2r)


def kernel(x, edge_index, batch, Wl1, bl1, Wr1, Wl2, bl2, Wr2, Wl3, bl3, Wr3,
           Wc, bc, W1, b1, W2, b2):
  f32 = jnp.float32
  xp = jnp.zeros((NP_, D), f32).at[:N].set(x)
  src, dst = edge_index[0], edge_index[1]
  npad = EP - E
  pad_src = N + (jnp.arange(npad, dtype=jnp.int32) % NPAD_ROWS)
  pad_dst = N + ((jnp.arange(npad, dtype=jnp.int32) * 7) % NPAD_ROWS)
  src2d = jnp.concatenate([src, pad_src]).reshape(EP // CH, CH)
  dst2d = jnp.concatenate([dst, pad_dst]).reshape(EP // CH, CH)
  zeros = jnp.zeros((NP_, D), f32)
  zcol = jnp.zeros((NP_,), f32)

  parts, degp = _agg_deg()(xp, src2d, dst2d, zeros, zcol)
  h1, inv = _layer1_tc(parts, degp.reshape(NC, NP_, 1), xp, Wl1, Wr1,
                       bl1.reshape(1, H))
  parts2 = _agg()(h1, src2d, dst2d, zeros, zcol)
  h2 = _layerN_tc(parts2, inv, h1, Wl2, Wr2, bl2.reshape(1, H))
  parts3 = _agg()(h2, src2d, dst2d, zeros, zcol)
  h3 = _layerN_tc(parts3, inv, h2, Wl3, Wr3, bl3.reshape(1, H))

  pooled = _sortpool()(h3, batch)

  WcT = jnp.transpose(Wc, (2, 1, 0)).reshape(KW * H, CONV_OUT)
  W1r = W1.reshape(CONV_OUT, T, H).transpose(1, 0, 2).reshape(T * CONV_OUT, H)
  return _head_tc(pooled.reshape(B, K * D), WcT, bc.reshape(1, CONV_OUT),
                  W1r, b1.reshape(1, H), W2, b2.reshape(1, C))


# speedup vs baseline: 1.2065x; 1.0014x over previous
import functools

import jax
import jax.numpy as jnp
from jax import lax
from jax.experimental import pallas as pl
from jax.experimental.pallas import tpu as pltpu
from jax.experimental.pallas import tpu_sc as plsc

N = 10000
E = 320000
D = 128
H = 128
B = 64
K = 30
C = 10
CONV_OUT = 32
KW = 5
T = K - KW + 1

NP_ = 10240
NPAD_ROWS = NP_ - N
NC, NS, L = 2, 16, 16
CH = 128
UNITS_PER_TILE = 80
EP = NC * NS * UNITS_PER_TILE * CH
ROWS_PER_TILE = NP_ // NS

NEG = -1.0e30


@functools.lru_cache(maxsize=None)
def _mesh():
  return plsc.VectorSubcoreMesh(
      core_axis_name="c", subcore_axis_name="s", num_cores=NC,
      num_subcores=NS)


def _make_agg(with_deg):
  out_type = jax.ShapeDtypeStruct((NC, NP_, D), jnp.float32)
  if with_deg:
    out_type = [out_type, jax.ShapeDtypeStruct((NC * NP_,), jnp.float32)]
  scratch = [
      pltpu.VMEM_SHARED((NP_, D), jnp.float32),
      pltpu.VMEM((UNITS_PER_TILE // 2, CH), jnp.int32),
      pltpu.VMEM((UNITS_PER_TILE // 2, CH), jnp.int32),
      pltpu.VMEM((CH, D), jnp.float32),
      pltpu.VMEM((CH, D), jnp.float32),
      pltpu.SemaphoreType.DMA,
      pltpu.SemaphoreType.DMA,
      pltpu.SemaphoreType.DMA,
      pltpu.SemaphoreType.DMA,
  ]
  if with_deg:
    scratch.insert(1, pltpu.VMEM_SHARED((NP_,), jnp.float32))
    scratch.append(pltpu.VMEM((CH,), jnp.float32))
    scratch.append(pltpu.SemaphoreType.DMA)

  def body(h_hbm, src_hbm, dst_hbm, zeros_hbm, zcol_hbm, *rest):
    if with_deg:
      (out_hbm, deg_hbm, acc, dega, src_v, dst_v, rows_a, rows_b,
       sem_a, sem_b, ssem_a, ssem_b, ones_v, dsem) = rest
    else:
      (out_hbm, acc, src_v, dst_v, rows_a, rows_b, sem_a, sem_b,
       ssem_a, ssem_b) = rest
    c = lax.axis_index("c")
    s = lax.axis_index("s")
    w = c * NS + s

    pltpu.sync_copy(zeros_hbm.at[pl.ds(s * ROWS_PER_TILE, ROWS_PER_TILE)],
                    acc.at[pl.ds(s * ROWS_PER_TILE, ROWS_PER_TILE)])
    if with_deg:
      pltpu.sync_copy(zcol_hbm.at[pl.ds(s * ROWS_PER_TILE, ROWS_PER_TILE)],
                      dega.at[pl.ds(s * ROWS_PER_TILE, ROWS_PER_TILE)])
      for j in range(CH // L):
        ones_v[pl.ds(j * L, L)] = jnp.full((L,), 1.0, jnp.float32)
    plsc.subcore_barrier()

    UPH = UNITS_PER_TILE // 2

    for ph in range(2):
      pltpu.sync_copy(src_hbm.at[pl.ds(w * UNITS_PER_TILE + ph * UPH, UPH)],
                      src_v)
      pltpu.sync_copy(dst_hbm.at[pl.ds(w * UNITS_PER_TILE + ph * UPH, UPH)],
                      dst_v)
      pltpu.async_copy(h_hbm.at[src_v.at[0]], rows_a, sem_a)

      def pair(p, carry):
        u0 = 2 * p
        pltpu.async_copy(h_hbm.at[src_v.at[u0 + 1]], rows_b, sem_b)
        pltpu.make_async_copy(h_hbm.at[src_v.at[u0]], rows_a, sem_a).wait()
        pltpu.sync_copy(rows_a, acc.at[dst_v.at[u0]], add=True)
        if with_deg:
          pltpu.async_copy(ones_v, dega.at[dst_v.at[u0]], dsem, add=True)

        @pl.when(p < UPH // 2 - 1)
        def _():
          pltpu.async_copy(h_hbm.at[src_v.at[u0 + 2]], rows_a, sem_a)

        pltpu.make_async_copy(h_hbm.at[src_v.at[u0 + 1]], rows_b,
                              sem_b).wait()
        pltpu.sync_copy(rows_b, acc.at[dst_v.at[u0 + 1]], add=True)
        if with_deg:
          pltpu.async_copy(ones_v, dega.at[dst_v.at[u0 + 1]], dsem,
                           add=True)
        return carry

      lax.fori_loop(0, UPH // 2, pair, 0)

      if with_deg:
        def deg_drain(i, carry):
          pltpu.make_async_copy(ones_v, dega.at[dst_v.at[0]], dsem).wait()
          return carry
        lax.fori_loop(0, UPH, deg_drain, 0)
    plsc.subcore_barrier()

    pltpu.sync_copy(acc.at[pl.ds(s * ROWS_PER_TILE, ROWS_PER_TILE)],
                    out_hbm.at[c, pl.ds(s * ROWS_PER_TILE, ROWS_PER_TILE)])
    if with_deg:
      pltpu.sync_copy(
          dega.at[pl.ds(s * ROWS_PER_TILE, ROWS_PER_TILE)],
          deg_hbm.at[pl.ds(c * NP_ + s * ROWS_PER_TILE, ROWS_PER_TILE)])

  return pl.kernel(body, out_type=out_type, mesh=_mesh(),
                   scratch_types=scratch,
                   compiler_params=pltpu.CompilerParams(
                       needs_layout_passes=False))


_agg_deg = functools.lru_cache(maxsize=None)(lambda: _make_agg(True))
_agg = functools.lru_cache(maxsize=None)(lambda: _make_agg(False))


_RB = 2048


def _linr_body(h, Wr, bl, out):
  out[...] = (jnp.dot(h[...], Wr[...], preferred_element_type=jnp.float32)
              + bl[...])


def _linr_tc(h, Wr, bl):
  return pl.pallas_call(
      _linr_body,
      grid=(NP_ // _RB,),
      in_specs=[
          pl.BlockSpec((_RB, D), lambda i: (i, 0)),
          pl.BlockSpec((D, H), lambda i: (0, 0)),
          pl.BlockSpec((1, H), lambda i: (0, 0)),
      ],
      out_specs=pl.BlockSpec((_RB, H), lambda i: (i, 0)),
      out_shape=jax.ShapeDtypeStruct((NP_, H), jnp.float32),
  )(h, Wr, bl)


def _layer1_body(parts, degp, hWr, Wl, out, inv_out):
  i = pl.program_id(0)
  d = degp[0] + degp[1]
  inv = 1.0 / jnp.maximum(d, 1.0)
  inv_out[...] = inv
  _layer_common(parts, inv, hWr, Wl, out, i)


def _layerN_body(parts, inv_ref, hWr, Wl, out):
  i = pl.program_id(0)
  _layer_common(parts, inv_ref[...], hWr, Wl, out, i)


def _layer_common(parts, inv, hWr, Wl, out, i):
  pp = parts[0] + parts[1]
  aggn = pp * inv
  hn = (jnp.dot(aggn, Wl[...], preferred_element_type=jnp.float32)
        + hWr[...])
  hn = jnp.maximum(hn, 0.0)
  rows = i * _RB + lax.broadcasted_iota(jnp.int32, (_RB, D), 0)
  out[...] = jnp.where(rows < N, hn, 0.0)


def _layer1_tc(parts, degp, hWr, Wl):
  grid = (NP_ // _RB,)
  return pl.pallas_call(
      _layer1_body,
      grid=grid,
      in_specs=[
          pl.BlockSpec((NC, _RB, D), lambda i: (0, i, 0)),
          pl.BlockSpec((NC, _RB, 1), lambda i: (0, i, 0)),
          pl.BlockSpec((_RB, D), lambda i: (i, 0)),
          pl.BlockSpec((D, H), lambda i: (0, 0)),
      ],
      out_specs=[
          pl.BlockSpec((_RB, H), lambda i: (i, 0)),
          pl.BlockSpec((_RB, 1), lambda i: (i, 0)),
      ],
      out_shape=[
          jax.ShapeDtypeStruct((NP_, H), jnp.float32),
          jax.ShapeDtypeStruct((NP_, 1), jnp.float32),
      ],
  )(parts, degp, hWr, Wl)


def _layerN_tc(parts, inv, hWr, Wl):
  grid = (NP_ // _RB,)
  return pl.pallas_call(
      _layerN_body,
      grid=grid,
      in_specs=[
          pl.BlockSpec((NC, _RB, D), lambda i: (0, i, 0)),
          pl.BlockSpec((_RB, 1), lambda i: (i, 0)),
          pl.BlockSpec((_RB, D), lambda i: (i, 0)),
          pl.BlockSpec((D, H), lambda i: (0, 0)),
      ],
      out_specs=pl.BlockSpec((_RB, H), lambda i: (i, 0)),
      out_shape=jax.ShapeDtypeStruct((NP_, H), jnp.float32),
  )(parts, inv, hWr, Wl)


CHKR = 128
NB_BATCH = N // L


def _sortpool_body(h_hbm, batch_hbm, pooled_hbm,
                   bat_v, keys_v, stage_v, selidx_v, outrows_v, cnt_v, sem):
  c = lax.axis_index("c")
  s = lax.axis_index("s")
  w = c * NS + s

  pltpu.sync_copy(batch_hbm, bat_v)
  iota = lax.iota(jnp.int32, L)

  for gl in range(2):
    g = w * 2 + gl

    zz = jnp.zeros((L,), jnp.int32)
    cnt_v[0, :] = zz
    cnt_v[1, :] = zz

    def cnt_body(i, carry):
      v = bat_v[pl.ds(i * L, L)]
      cnt_v[0, :] = cnt_v[0, :] + jnp.where(v < g, 1, 0)
      cnt_v[1, :] = cnt_v[1, :] + jnp.where(v == g, 1, 0)
      return carry

    lax.fori_loop(0, NB_BATCH, cnt_body, 0)
    start = jnp.sum(cnt_v[0, :])
    cnt = jnp.sum(cnt_v[1, :])

    abase = pl.multiple_of((start // 8) * 8, 8)
    off = start - abase
    total = off + cnt
    nchunks = (total + CHKR - 1) // CHKR
    col16 = jnp.full((L,), D - 1, jnp.int32)

    def chunk_body(j, carry):
      pltpu.sync_copy(
          h_hbm.at[pl.ds(pl.multiple_of(abase + j * CHKR, 8), CHKR)], stage_v)
      for jj in range(CHKR // L):
        rows16 = iota + jj * L
        kv = plsc.load_gather(stage_v, [rows16, col16])
        pos = j * CHKR + jj * L + iota
        kv = jnp.where((pos >= off) & (pos < total), kv, NEG)
        keys_v[pl.ds(j * CHKR + jj * L, L)] = kv
      return carry

    lax.fori_loop(0, nchunks, chunk_body, 0)

    nv = (total + L - 1) // L

    selidx_v[pl.ds(gl * 2 * L, L)] = N + ((w * 29 + iota) % NPAD_ROWS)
    selidx_v[pl.ds(gl * 2 * L + L, L)] = N + ((w * 29 + 101 + iota)
                                              % NPAD_ROWS)

    def sel_body(k, carry):
      def mx_body(i, m):
        return jnp.maximum(m, jnp.max(keys_v[pl.ds(i * L, L)]))
      m = lax.fori_loop(0, nv, mx_body, jnp.float32(NEG))

      def fi_body(i, f):
        v = keys_v[pl.ds(i * L, L)]
        cand = jnp.where(v == m, iota + i * L, jnp.int32(2**30))
        return jnp.minimum(f, jnp.min(cand))
      found = lax.fori_loop(0, nv, fi_body, jnp.int32(2**30))

      node = jnp.where(k < cnt, abase + found,
                       N + ((w * 53 + k * 3) % NPAD_ROWS))
      plsc.store_scatter(keys_v, [jnp.full((L,), found, jnp.int32)],
                         jnp.full((L,), NEG, jnp.float32),
                         mask=(iota == 0))
      plsc.store_scatter(selidx_v, [jnp.full((L,), gl * 2 * L + k,
                                             jnp.int32)],
                         jnp.full((L,), node, jnp.int32),
                         mask=(iota == 0))
      return carry

    lax.fori_loop(0, K, sel_body, 0)

    pltpu.async_copy(h_hbm.at[selidx_v.at[pl.ds(gl * 2 * L, 2 * L)]],
                     outrows_v, sem).wait()
    pltpu.sync_copy(outrows_v.at[pl.ds(0, K)], pooled_hbm.at[g])


@functools.lru_cache(maxsize=None)
def _sortpool():
  return pl.kernel(
      _sortpool_body,
      out_type=jax.ShapeDtypeStruct((B, K, D), jnp.float32),
      mesh=_mesh(),
      scratch_types=[
          pltpu.VMEM((N,), jnp.int32),
          pltpu.VMEM((NP_,), jnp.float32),
          pltpu.VMEM((CHKR, D), jnp.float32),
          pltpu.VMEM((2 * 2 * L,), jnp.int32),
          pltpu.VMEM((2 * L, D), jnp.float32),
          pltpu.VMEM((2, L), jnp.int32),
          pltpu.SemaphoreType.DMA,
      ],
      compiler_params=pltpu.CompilerParams(needs_layout_passes=False),
  )


def _head_body(p2, WcT, bc2, W1r, b1r, W2, b2r, out):
  h1acc = jnp.zeros((B, H), jnp.float32)
  for t in range(T):
    zt = jnp.zeros((B, CONV_OUT), jnp.float32)
    for kw in range(KW):
      zt = zt + jnp.dot(p2[:, (t + kw) * H:(t + kw + 1) * H],
                        WcT[kw * H:(kw + 1) * H, :],
                        preferred_element_type=jnp.float32)
    a = jnp.maximum(zt + bc2[...], 0.0)
    h1acc = h1acc + jnp.dot(a, W1r[t * CONV_OUT:(t + 1) * CONV_OUT, :],
                            preferred_element_type=jnp.float32)
  h1 = jnp.maximum(h1acc + b1r[...], 0.0)
  logits = jnp.dot(h1, W2[...], preferred_element_type=jnp.float32) + b2r[...]
  m = jnp.max(logits, axis=-1, keepdims=True)
  out[...] = logits - m - jnp.log(
      jnp.sum(jnp.exp(logits - m), axis=-1, keepdims=True))


def _head_tc(p2, WcT, bc2, W1r, b1r, W2, b2r):
  return pl.pallas_call(
      _head_body,
      out_shape=jax.ShapeDtypeStruct((B, C), jnp.float32),
  )(p2, WcT, bc2, W1r, b1r, W2, b2r)


def kernel(x, edge_index, batch, Wl1, bl1, Wr1, Wl2, bl2, Wr2, Wl3, bl3, Wr3,
           Wc, bc, W1, b1, W2, b2):
  f32 = jnp.float32
  xp = jnp.zeros((NP_, D), f32).at[:N].set(x)
  src, dst = edge_index[0], edge_index[1]
  npad = EP - E
  pad_src = N + (jnp.arange(npad, dtype=jnp.int32) % NPAD_ROWS)
  pad_dst = N + ((jnp.arange(npad, dtype=jnp.int32) * 7) % NPAD_ROWS)
  src2d = jnp.concatenate([src, pad_src]).reshape(EP // CH, CH)
  dst2d = jnp.concatenate([dst, pad_dst]).reshape(EP // CH, CH)
  zeros = jnp.zeros((NP_, D), f32)
  zcol = jnp.zeros((NP_,), f32)

  hWr1 = _linr_tc(xp, Wr1, bl1.reshape(1, H))
  parts, degp = _agg_deg()(xp, src2d, dst2d, zeros, zcol)
  h1, inv = _layer1_tc(parts, degp.reshape(NC, NP_, 1), hWr1, Wl1)
  hWr2 = _linr_tc(h1, Wr2, bl2.reshape(1, H))
  parts2 = _agg()(h1, src2d, dst2d, zeros, zcol)
  h2 = _layerN_tc(parts2, inv, hWr2, Wl2)
  hWr3 = _linr_tc(h2, Wr3, bl3.reshape(1, H))
  parts3 = _agg()(h2, src2d, dst2d, zeros, zcol)
  h3 = _layerN_tc(parts3, inv, hWr3, Wl3)

  pooled = _sortpool()(h3, batch)

  WcT = jnp.transpose(Wc, (2, 1, 0)).reshape(KW * H, CONV_OUT)
  W1r = W1.reshape(CONV_OUT, T, H).transpose(1, 0, 2).reshape(T * CONV_OUT, H)
  return _head_tc(pooled.reshape(B, K * D), WcT, bc.reshape(1, CONV_OUT),
                  W1r, b1.reshape(1, H), W2, b2.reshape(1, C))


# speedup vs baseline: 1.2234x; 1.0141x over previous
import functools

import jax
import jax.numpy as jnp
from jax import lax
from jax.experimental import pallas as pl
from jax.experimental.pallas import tpu as pltpu
from jax.experimental.pallas import tpu_sc as plsc

N = 10000
E = 320000
D = 128
H = 128
B = 64
K = 30
C = 10
CONV_OUT = 32
KW = 5
T = K - KW + 1

NP_ = 10240
NPAD_ROWS = NP_ - N
NC, NS, L = 2, 16, 16
CH = 128
UNITS_PER_TILE = 80
EP = NC * NS * UNITS_PER_TILE * CH
ROWS_PER_TILE = NP_ // NS

NEG = -1.0e30


@functools.lru_cache(maxsize=None)
def _mesh():
  return plsc.VectorSubcoreMesh(
      core_axis_name="c", subcore_axis_name="s", num_cores=NC,
      num_subcores=NS)


def _make_agg(with_deg):
  out_type = jax.ShapeDtypeStruct((NC, NP_, D), jnp.float32)
  if with_deg:
    out_type = [out_type, jax.ShapeDtypeStruct((NC * NP_,), jnp.float32)]
  scratch = [
      pltpu.VMEM_SHARED((NP_, D), jnp.float32),
      pltpu.VMEM((UNITS_PER_TILE // 2, CH), jnp.int32),
      pltpu.VMEM((UNITS_PER_TILE // 2, CH), jnp.int32),
      pltpu.VMEM((CH, D), jnp.float32),
      pltpu.VMEM((CH, D), jnp.float32),
      pltpu.SemaphoreType.DMA,
      pltpu.SemaphoreType.DMA,
      pltpu.SemaphoreType.DMA,
      pltpu.SemaphoreType.DMA,
  ]
  if with_deg:
    scratch.insert(1, pltpu.VMEM_SHARED((NP_,), jnp.float32))
    scratch.append(pltpu.VMEM((CH,), jnp.float32))
    scratch.append(pltpu.SemaphoreType.DMA)

  def body(h_hbm, src_hbm, dst_hbm, zeros_hbm, *rest):
    if with_deg:
      (zcol_hbm, ones_hbm, out_hbm, deg_hbm, acc, dega, src_v, dst_v,
       rows_a, rows_b, sem_a, sem_b, ssem_a, ssem_b, ones_v, dsem) = rest
    else:
      (out_hbm, acc, src_v, dst_v, rows_a, rows_b, sem_a, sem_b,
       ssem_a, ssem_b) = rest
    c = lax.axis_index("c")
    s = lax.axis_index("s")
    w = c * NS + s

    pltpu.sync_copy(zeros_hbm.at[pl.ds(s * ROWS_PER_TILE, ROWS_PER_TILE)],
                    acc.at[pl.ds(s * ROWS_PER_TILE, ROWS_PER_TILE)])
    if with_deg:
      pltpu.sync_copy(zcol_hbm.at[pl.ds(s * ROWS_PER_TILE, ROWS_PER_TILE)],
                      dega.at[pl.ds(s * ROWS_PER_TILE, ROWS_PER_TILE)])
      pltpu.sync_copy(ones_hbm, ones_v)
    plsc.subcore_barrier()

    UPH = UNITS_PER_TILE // 2

    for ph in range(2):
      pltpu.sync_copy(src_hbm.at[pl.ds(w * UNITS_PER_TILE + ph * UPH, UPH)],
                      src_v)
      pltpu.sync_copy(dst_hbm.at[pl.ds(w * UNITS_PER_TILE + ph * UPH, UPH)],
                      dst_v)
      pltpu.async_copy(h_hbm.at[src_v.at[0]], rows_a, sem_a)

      def pair(p, carry):
        u0 = 2 * p
        pltpu.async_copy(h_hbm.at[src_v.at[u0 + 1]], rows_b, sem_b)
        pltpu.make_async_copy(h_hbm.at[src_v.at[u0]], rows_a, sem_a).wait()
        pltpu.sync_copy(rows_a, acc.at[dst_v.at[u0]], add=True)
        if with_deg:
          pltpu.async_copy(ones_v, dega.at[dst_v.at[u0]], dsem, add=True)

        @pl.when(p < UPH // 2 - 1)
        def _():
          pltpu.async_copy(h_hbm.at[src_v.at[u0 + 2]], rows_a, sem_a)

        pltpu.make_async_copy(h_hbm.at[src_v.at[u0 + 1]], rows_b,
                              sem_b).wait()
        pltpu.sync_copy(rows_b, acc.at[dst_v.at[u0 + 1]], add=True)
        if with_deg:
          pltpu.async_copy(ones_v, dega.at[dst_v.at[u0 + 1]], dsem,
                           add=True)
        return carry

      lax.fori_loop(0, UPH // 2, pair, 0)

      if with_deg:
        def deg_drain(i, carry):
          pltpu.make_async_copy(ones_v, dega.at[dst_v.at[0]], dsem).wait()
          return carry
        lax.fori_loop(0, UPH, deg_drain, 0)
    plsc.subcore_barrier()

    pltpu.sync_copy(acc.at[pl.ds(s * ROWS_PER_TILE, ROWS_PER_TILE)],
                    out_hbm.at[c, pl.ds(s * ROWS_PER_TILE, ROWS_PER_TILE)])
    if with_deg:
      pltpu.sync_copy(
          dega.at[pl.ds(s * ROWS_PER_TILE, ROWS_PER_TILE)],
          deg_hbm.at[pl.ds(c * NP_ + s * ROWS_PER_TILE, ROWS_PER_TILE)])

  return pl.kernel(body, out_type=out_type, mesh=_mesh(),
                   scratch_types=scratch,
                   compiler_params=pltpu.CompilerParams(
                       needs_layout_passes=False))


_agg_deg = functools.lru_cache(maxsize=None)(lambda: _make_agg(True))
_agg = functools.lru_cache(maxsize=None)(lambda: _make_agg(False))


_RB = 2048


def _linr_body(h, Wr, bl, out):
  out[...] = (jnp.dot(h[...], Wr[...], preferred_element_type=jnp.float32)
              + bl[...])


def _linr_tc(h, Wr, bl):
  return pl.pallas_call(
      _linr_body,
      grid=(NP_ // _RB,),
      in_specs=[
          pl.BlockSpec((_RB, D), lambda i: (i, 0)),
          pl.BlockSpec((D, H), lambda i: (0, 0)),
          pl.BlockSpec((1, H), lambda i: (0, 0)),
      ],
      out_specs=pl.BlockSpec((_RB, H), lambda i: (i, 0)),
      out_shape=jax.ShapeDtypeStruct((NP_, H), jnp.float32),
  )(h, Wr, bl)


def _layer1_body(parts, degp, hWr, Wl, out, inv_out):
  i = pl.program_id(0)
  d = degp[0] + degp[1]
  inv = 1.0 / jnp.maximum(d, 1.0)
  inv_out[...] = inv
  _layer_common(parts, inv, hWr, Wl, out, i)


def _layerN_body(parts, inv_ref, hWr, Wl, out):
  i = pl.program_id(0)
  _layer_common(parts, inv_ref[...], hWr, Wl, out, i)


def _layer_common(parts, inv, hWr, Wl, out, i):
  pp = parts[0] + parts[1]
  aggn = pp * inv
  hn = (jnp.dot(aggn, Wl[...], preferred_element_type=jnp.float32)
        + hWr[...])
  hn = jnp.maximum(hn, 0.0)
  rows = i * _RB + lax.broadcasted_iota(jnp.int32, (_RB, D), 0)
  out[...] = jnp.where(rows < N, hn, 0.0)


def _layer1_tc(parts, degp, hWr, Wl):
  grid = (NP_ // _RB,)
  return pl.pallas_call(
      _layer1_body,
      grid=grid,
      in_specs=[
          pl.BlockSpec((NC, _RB, D), lambda i: (0, i, 0)),
          pl.BlockSpec((NC, _RB, 1), lambda i: (0, i, 0)),
          pl.BlockSpec((_RB, D), lambda i: (i, 0)),
          pl.BlockSpec((D, H), lambda i: (0, 0)),
      ],
      out_specs=[
          pl.BlockSpec((_RB, H), lambda i: (i, 0)),
          pl.BlockSpec((_RB, 1), lambda i: (i, 0)),
      ],
      out_shape=[
          jax.ShapeDtypeStruct((NP_, H), jnp.float32),
          jax.ShapeDtypeStruct((NP_, 1), jnp.float32),
      ],
  )(parts, degp, hWr, Wl)


def _layerN_tc(parts, inv, hWr, Wl):
  grid = (NP_ // _RB,)
  return pl.pallas_call(
      _layerN_body,
      grid=grid,
      in_specs=[
          pl.BlockSpec((NC, _RB, D), lambda i: (0, i, 0)),
          pl.BlockSpec((_RB, 1), lambda i: (i, 0)),
          pl.BlockSpec((_RB, D), lambda i: (i, 0)),
          pl.BlockSpec((D, H), lambda i: (0, 0)),
      ],
      out_specs=pl.BlockSpec((_RB, H), lambda i: (i, 0)),
      out_shape=jax.ShapeDtypeStruct((NP_, H), jnp.float32),
  )(parts, inv, hWr, Wl)


CHKR = 128
NB_BATCH = N // L


def _sortpool_body(h_hbm, batch_hbm, pooled_hbm,
                   bat_v, keys_v, stage_v, selidx_v, outrows_v, cnt_v,
                   vmax_v, sem):
  c = lax.axis_index("c")
  s = lax.axis_index("s")
  w = c * NS + s

  pltpu.sync_copy(batch_hbm, bat_v)
  iota = lax.iota(jnp.int32, L)

  g0 = w * 2
  zz = jnp.zeros((L,), jnp.int32)
  cnt_v[0, :] = zz
  cnt_v[1, :] = zz
  cnt_v[2, :] = zz

  def cnt_body(i, carry):
    v = bat_v[pl.ds(i * L, L)]
    cnt_v[0, :] = cnt_v[0, :] + jnp.where(v < g0, 1, 0)
    cnt_v[1, :] = cnt_v[1, :] + jnp.where(v == g0, 1, 0)
    cnt_v[2, :] = cnt_v[2, :] + jnp.where(v == g0 + 1, 1, 0)
    return carry

  lax.fori_loop(0, NB_BATCH, cnt_body, 0)
  start0 = jnp.sum(cnt_v[0, :])
  cnt0 = jnp.sum(cnt_v[1, :])
  cnt1 = jnp.sum(cnt_v[2, :])

  for gl in range(2):
    g = w * 2 + gl
    start = start0 if gl == 0 else start0 + cnt0
    cnt = cnt0 if gl == 0 else cnt1

    abase = pl.multiple_of((start // 8) * 8, 8)
    off = start - abase
    total = off + cnt
    nchunks = (total + CHKR - 1) // CHKR
    col16 = jnp.full((L,), D - 1, jnp.int32)

    def chunk_body(j, carry):
      pltpu.sync_copy(
          h_hbm.at[pl.ds(pl.multiple_of(abase + j * CHKR, 8), CHKR)], stage_v)
      for jj in range(CHKR // L):
        rows16 = iota + jj * L
        kv = plsc.load_gather(stage_v, [rows16, col16])
        pos = j * CHKR + jj * L + iota
        kv = jnp.where((pos >= off) & (pos < total), kv, NEG)
        keys_v[pl.ds(j * CHKR + jj * L, L)] = kv
      return carry

    lax.fori_loop(0, nchunks, chunk_body, 0)

    nv = (total + L - 1) // L
    nvv = (nv + L - 1) // L

    def vm_body(i, carry):
      mv = jnp.max(keys_v[pl.ds(i * L, L)])
      plsc.store_scatter(vmax_v, [jnp.full((L,), i, jnp.int32)],
                         jnp.full((L,), mv, jnp.float32),
                         mask=(iota == 0))
      return carry

    lax.fori_loop(0, nv, vm_body, 0)

    selidx_v[pl.ds(gl * 2 * L, L)] = N + ((w * 29 + iota) % NPAD_ROWS)
    selidx_v[pl.ds(gl * 2 * L + L, L)] = N + ((w * 29 + 101 + iota)
                                              % NPAD_ROWS)

    def sel_body(k, carry):
      def mx_body(i, m):
        pos = iota + i * L
        vv = jnp.where(pos < nv, vmax_v[pl.ds(i * L, L)], NEG)
        return jnp.maximum(m, jnp.max(vv))
      m = lax.fori_loop(0, nvv, mx_body, jnp.float32(NEG))

      def fv_body(i, f):
        pos = iota + i * L
        vv = jnp.where(pos < nv, vmax_v[pl.ds(i * L, L)], NEG)
        cand = jnp.where(vv == m, pos, jnp.int32(2**30))
        return jnp.minimum(f, jnp.min(cand))
      vi = lax.fori_loop(0, nvv, fv_body, jnp.int32(2**30))
      vi = jnp.minimum(vi, jnp.int32(NP_ // L - 1))

      kv = keys_v[pl.ds(vi * L, L)]
      cand2 = jnp.where(kv == m, iota + vi * L, jnp.int32(2**30))
      found = jnp.min(cand2)

      node = jnp.where(k < cnt, abase + found,
                       N + ((w * 53 + k * 3) % NPAD_ROWS))
      kv2 = jnp.where(iota + vi * L == found, NEG, kv)
      keys_v[pl.ds(vi * L, L)] = kv2
      plsc.store_scatter(vmax_v, [jnp.full((L,), vi, jnp.int32)],
                         jnp.full((L,), jnp.max(kv2), jnp.float32),
                         mask=(iota == 0))
      plsc.store_scatter(selidx_v, [jnp.full((L,), gl * 2 * L + k,
                                             jnp.int32)],
                         jnp.full((L,), node, jnp.int32),
                         mask=(iota == 0))
      return carry

    lax.fori_loop(0, K, sel_body, 0)

    pltpu.async_copy(h_hbm.at[selidx_v.at[pl.ds(gl * 2 * L, 2 * L)]],
                     outrows_v, sem).wait()
    pltpu.sync_copy(outrows_v.at[pl.ds(0, K)], pooled_hbm.at[g])


@functools.lru_cache(maxsize=None)
def _sortpool():
  return pl.kernel(
      _sortpool_body,
      out_type=jax.ShapeDtypeStruct((B, K, D), jnp.float32),
      mesh=_mesh(),
      scratch_types=[
          pltpu.VMEM((N,), jnp.int32),
          pltpu.VMEM((NP_,), jnp.float32),
          pltpu.VMEM((CHKR, D), jnp.float32),
          pltpu.VMEM((2 * 2 * L,), jnp.int32),
          pltpu.VMEM((2 * L, D), jnp.float32),
          pltpu.VMEM((3, L), jnp.int32),
          pltpu.VMEM((NP_ // L,), jnp.float32),
          pltpu.SemaphoreType.DMA,
      ],
      compiler_params=pltpu.CompilerParams(needs_layout_passes=False),
  )


def _head_body(p2, WcT, bc2, W1r, b1r, W2, b2r, out):
  h1acc = jnp.zeros((B, H), jnp.float32)
  for t in range(T):
    zt = jnp.zeros((B, CONV_OUT), jnp.float32)
    for kw in range(KW):
      zt = zt + jnp.dot(p2[:, (t + kw) * H:(t + kw + 1) * H],
                        WcT[kw * H:(kw + 1) * H, :],
                        preferred_element_type=jnp.float32)
    a = jnp.maximum(zt + bc2[...], 0.0)
    h1acc = h1acc + jnp.dot(a, W1r[t * CONV_OUT:(t + 1) * CONV_OUT, :],
                            preferred_element_type=jnp.float32)
  h1 = jnp.maximum(h1acc + b1r[...], 0.0)
  logits = jnp.dot(h1, W2[...], preferred_element_type=jnp.float32) + b2r[...]
  m = jnp.max(logits, axis=-1, keepdims=True)
  out[...] = logits - m - jnp.log(
      jnp.sum(jnp.exp(logits - m), axis=-1, keepdims=True))


def _head_tc(p2, WcT, bc2, W1r, b1r, W2, b2r):
  return pl.pallas_call(
      _head_body,
      out_shape=jax.ShapeDtypeStruct((B, C), jnp.float32),
  )(p2, WcT, bc2, W1r, b1r, W2, b2r)


def kernel(x, edge_index, batch, Wl1, bl1, Wr1, Wl2, bl2, Wr2, Wl3, bl3, Wr3,
           Wc, bc, W1, b1, W2, b2):
  f32 = jnp.float32
  xp = jnp.zeros((NP_, D), f32).at[:N].set(x)
  src, dst = edge_index[0], edge_index[1]
  npad = EP - E
  pad_src = N + (jnp.arange(npad, dtype=jnp.int32) % NPAD_ROWS)
  pad_dst = N + ((jnp.arange(npad, dtype=jnp.int32) * 7) % NPAD_ROWS)
  src2d = jnp.concatenate([src, pad_src]).reshape(EP // CH, CH)
  dst2d = jnp.concatenate([dst, pad_dst]).reshape(EP // CH, CH)
  zeros = jnp.zeros((NP_, D), f32)
  zcol = jnp.zeros((NP_,), f32)
  ones1 = jnp.ones((CH,), f32)

  hWr1 = _linr_tc(xp, Wr1, bl1.reshape(1, H))
  parts, degp = _agg_deg()(xp, src2d, dst2d, zeros, zcol, ones1)
  h1, inv = _layer1_tc(parts, degp.reshape(NC, NP_, 1), hWr1, Wl1)
  hWr2 = _linr_tc(h1, Wr2, bl2.reshape(1, H))
  parts2 = _agg()(h1, src2d, dst2d, zeros)
  h2 = _layerN_tc(parts2, inv, hWr2, Wl2)
  hWr3 = _linr_tc(h2, Wr3, bl3.reshape(1, H))
  parts3 = _agg()(h2, src2d, dst2d, zeros)
  h3 = _layerN_tc(parts3, inv, hWr3, Wl3)

  pooled = _sortpool()(h3, batch)

  WcT = jnp.transpose(Wc, (2, 1, 0)).reshape(KW * H, CONV_OUT)
  W1r = W1.reshape(CONV_OUT, T, H).transpose(1, 0, 2).reshape(T * CONV_OUT, H)
  return _head_tc(pooled.reshape(B, K * D), WcT, bc.reshape(1, CONV_OUT),
                  W1r, b1.reshape(1, H), W2, b2.reshape(1, C))
